# Initial kernel scaffold; baseline (speedup 1.0000x reference)
import functools

import jax
import jax.numpy as jnp
from jax import lax
from jax.experimental import pallas as pl
from jax.experimental.pallas import tpu as pltpu
from jax.experimental.pallas import tpu_sc as plsc

BATCH = 16384
D = 128


_info = plsc.get_sparse_core_info()
_NC, _NS = _info.num_cores, _info.num_subcores
_NW = _NC * _NS
_BPW = BATCH // _NW
_CH = 128
_NCHUNK = _BPW // _CH

_sc_mesh = plsc.VectorSubcoreMesh(core_axis_name="c", subcore_axis_name="s")


@functools.partial(
    pl.kernel,
    mesh=_sc_mesh,
    out_type=[
        jax.ShapeDtypeStruct((BATCH, D), jnp.float32),
        jax.ShapeDtypeStruct((BATCH, D), jnp.float32),
    ],
    scratch_types=[
        pltpu.VMEM((_NCHUNK, _CH), jnp.int32),
        pltpu.VMEM((_NCHUNK, _CH), jnp.int32),
        pltpu.VMEM((_CH, D), jnp.float32),
        pltpu.SemaphoreType.DMA,
    ],
)
def _sc_gather(users_hbm, movies_hbm, eu_hbm, em_hbm, ue_out, me_out,
               uidx_v, midx_v, rows_v, sem):
    wid = lax.axis_index("s") * _NC + lax.axis_index("c")
    base = wid * _BPW
    pltpu.sync_copy(users_hbm.at[pl.ds(wid * _NCHUNK, _NCHUNK)], uidx_v)
    pltpu.sync_copy(movies_hbm.at[pl.ds(wid * _NCHUNK, _NCHUNK)], midx_v)
    for j in range(_NCHUNK):
        pltpu.async_copy(eu_hbm.at[uidx_v.at[j]], rows_v, sem).wait()
        pltpu.sync_copy(rows_v, ue_out.at[pl.ds(base + j * _CH, _CH)])
    for j in range(_NCHUNK):
        pltpu.async_copy(em_hbm.at[midx_v.at[j]], rows_v, sem).wait()
        pltpu.sync_copy(rows_v, me_out.at[pl.ds(base + j * _CH, _CH)])


_BM = 1024


def _mlp_body(ue_ref, me_ref, w1a_ref, w1b_ref, b1_ref, w2_ref, b2_ref,
              wf_ref, bf_ref, out_ref):
    x = jnp.dot(ue_ref[...], w1a_ref[...], preferred_element_type=jnp.float32)
    x = x + jnp.dot(me_ref[...], w1b_ref[...], preferred_element_type=jnp.float32)
    x = jax.nn.relu(x + b1_ref[...])
    x = jnp.dot(x, w2_ref[...], preferred_element_type=jnp.float32)
    x = jax.nn.relu(x + b2_ref[...])
    x = jnp.dot(x, wf_ref[...], preferred_element_type=jnp.float32)
    out_ref[...] = jax.nn.sigmoid(x + bf_ref[...])


def _mlp(ue, me, W1, b1, W2, b2, Wf, bf):
    h1, h2 = W1.shape[1], W2.shape[1]
    grid = (BATCH // _BM,)
    return pl.pallas_call(
        _mlp_body,
        grid=grid,
        in_specs=[
            pl.BlockSpec((_BM, D), lambda i: (i, 0)),
            pl.BlockSpec((_BM, D), lambda i: (i, 0)),
            pl.BlockSpec((D, h1), lambda i: (0, 0)),
            pl.BlockSpec((D, h1), lambda i: (0, 0)),
            pl.BlockSpec((1, h1), lambda i: (0, 0)),
            pl.BlockSpec((h1, h2), lambda i: (0, 0)),
            pl.BlockSpec((1, h2), lambda i: (0, 0)),
            pl.BlockSpec((h2, 1), lambda i: (0, 0)),
            pl.BlockSpec((1, 1), lambda i: (0, 0)),
        ],
        out_specs=pl.BlockSpec((_BM, 1), lambda i: (i, 0)),
        out_shape=jax.ShapeDtypeStruct((BATCH, 1), jnp.float32),
    )(ue, me, W1[:D], W1[D:], b1.reshape(1, h1), W2, b2.reshape(1, h2),
      Wf, bf.reshape(1, 1))


def kernel(users, movies, Eu, Em, W1, b1, W2, b2, Wf, bf):
    u2 = users.astype(jnp.int32).reshape(BATCH // _CH, _CH)
    m2 = movies.astype(jnp.int32).reshape(BATCH // _CH, _CH)
    ue, me = _sc_gather(u2, m2, Eu, Em)
    return _mlp(ue, me, W1, b1, W2, b2, Wf, bf)


# speedup vs baseline: 2.1498x; 2.1498x over previous
import functools

import jax
import jax.numpy as jnp
from jax import lax
from jax.experimental import pallas as pl
from jax.experimental.pallas import tpu as pltpu
from jax.experimental.pallas import tpu_sc as plsc

BATCH = 16384
D = 128


_NC, _NS = 2, 16
_NW = _NC * _NS
_BPW = BATCH // _NW
_CH = 128
_NCHUNK = _BPW // _CH

@functools.lru_cache(maxsize=1)
def _make_sc_gather():
    mesh = plsc.VectorSubcoreMesh(
        core_axis_name="c", subcore_axis_name="s",
        num_cores=_NC, num_subcores=_NS)

    @functools.partial(
        pl.kernel,
        mesh=mesh,
        out_type=[
            jax.ShapeDtypeStruct((BATCH, D), jnp.float32),
            jax.ShapeDtypeStruct((BATCH, D), jnp.float32),
        ],
        scratch_types=[
            pltpu.VMEM((_NCHUNK, _CH), jnp.int32),
            pltpu.VMEM((_NCHUNK, _CH), jnp.int32),
            pltpu.VMEM((_CH, D), jnp.float32),
            pltpu.SemaphoreType.DMA,
        ],
    )
    def _sc_gather(users_hbm, movies_hbm, eu_hbm, em_hbm, ue_out, me_out,
                   uidx_v, midx_v, rows_v, sem):
        wid = lax.axis_index("s") * _NC + lax.axis_index("c")
        base = wid * _BPW
        pltpu.sync_copy(users_hbm.at[pl.ds(wid * _NCHUNK, _NCHUNK)], uidx_v)
        pltpu.sync_copy(movies_hbm.at[pl.ds(wid * _NCHUNK, _NCHUNK)], midx_v)
        for j in range(_NCHUNK):
            pltpu.async_copy(eu_hbm.at[uidx_v.at[j]], rows_v, sem).wait()
            pltpu.sync_copy(rows_v, ue_out.at[pl.ds(base + j * _CH, _CH)])
        for j in range(_NCHUNK):
            pltpu.async_copy(em_hbm.at[midx_v.at[j]], rows_v, sem).wait()
            pltpu.sync_copy(rows_v, me_out.at[pl.ds(base + j * _CH, _CH)])

    return _sc_gather


_BM = 1024


def _mlp_body(ue_ref, me_ref, w1a_ref, w1b_ref, b1_ref, w2_ref, b2_ref,
              wf_ref, bf_ref, out_ref):
    x = jnp.dot(ue_ref[...], w1a_ref[...], preferred_element_type=jnp.float32)
    x = x + jnp.dot(me_ref[...], w1b_ref[...], preferred_element_type=jnp.float32)
    x = jax.nn.relu(x + b1_ref[...])
    x = jnp.dot(x, w2_ref[...], preferred_element_type=jnp.float32)
    x = jax.nn.relu(x + b2_ref[...])
    x = jnp.dot(x, wf_ref[...], preferred_element_type=jnp.float32)
    out_ref[...] = jax.nn.sigmoid(x + bf_ref[...])


def _mlp(ue, me, W1, b1, W2, b2, Wf, bf):
    h1, h2 = W1.shape[1], W2.shape[1]
    grid = (BATCH // _BM,)
    return pl.pallas_call(
        _mlp_body,
        grid=grid,
        in_specs=[
            pl.BlockSpec((_BM, D), lambda i: (i, 0)),
            pl.BlockSpec((_BM, D), lambda i: (i, 0)),
            pl.BlockSpec((D, h1), lambda i: (0, 0)),
            pl.BlockSpec((D, h1), lambda i: (0, 0)),
            pl.BlockSpec((1, h1), lambda i: (0, 0)),
            pl.BlockSpec((h1, h2), lambda i: (0, 0)),
            pl.BlockSpec((1, h2), lambda i: (0, 0)),
            pl.BlockSpec((h2, 1), lambda i: (0, 0)),
            pl.BlockSpec((1, 1), lambda i: (0, 0)),
        ],
        out_specs=pl.BlockSpec((_BM, 1), lambda i: (i, 0)),
        out_shape=jax.ShapeDtypeStruct((BATCH, 1), jnp.float32),
    )(ue, me, W1[:D], W1[D:], b1.reshape(1, h1), W2, b2.reshape(1, h2),
      Wf, bf.reshape(1, 1))


def kernel(users, movies, Eu, Em, W1, b1, W2, b2, Wf, bf):
    u2 = users.astype(jnp.int32).reshape(BATCH // _CH, _CH)
    m2 = movies.astype(jnp.int32).reshape(BATCH // _CH, _CH)
    ue, me = _make_sc_gather()(u2, m2, Eu, Em)
    return _mlp(ue, me, W1, b1, W2, b2, Wf, bf)


# speedup vs baseline: 2.1956x; 1.0213x over previous
import functools

import jax
import jax.numpy as jnp
from jax import lax
from jax.experimental import pallas as pl
from jax.experimental.pallas import tpu as pltpu
from jax.experimental.pallas import tpu_sc as plsc

BATCH = 16384
D = 128


_NC, _NS = 2, 16
_NW = _NC * _NS
_BPW = BATCH // _NW
_CH = 128
_NCHUNK = _BPW // _CH

@functools.lru_cache(maxsize=1)
def _make_sc_gather():
    mesh = plsc.VectorSubcoreMesh(
        core_axis_name="c", subcore_axis_name="s",
        num_cores=_NC, num_subcores=_NS)

    @functools.partial(
        pl.kernel,
        mesh=mesh,
        out_type=[
            jax.ShapeDtypeStruct((BATCH, D), jnp.float32),
            jax.ShapeDtypeStruct((BATCH, D), jnp.float32),
        ],
        scratch_types=[
            pltpu.VMEM((_NCHUNK, _CH), jnp.int32),
            pltpu.VMEM((_NCHUNK, _CH), jnp.int32),
            pltpu.VMEM((_CH, D), jnp.float32),
            pltpu.VMEM((_CH, D), jnp.float32),
            pltpu.SemaphoreType.DMA,
            pltpu.SemaphoreType.DMA,
        ],
    )
    def _sc_gather(users_hbm, movies_hbm, eu_hbm, em_hbm, ue_out, me_out,
                   uidx_v, midx_v, rows0_v, rows1_v, sem0, sem1):
        wid = lax.axis_index("s") * _NC + lax.axis_index("c")
        base = wid * _BPW
        pltpu.sync_copy(users_hbm.at[pl.ds(wid * _NCHUNK, _NCHUNK)], uidx_v)
        pltpu.sync_copy(movies_hbm.at[pl.ds(wid * _NCHUNK, _NCHUNK)], midx_v)
        chunks = (
            [(eu_hbm, uidx_v.at[j], ue_out, base + j * _CH)
             for j in range(_NCHUNK)]
            + [(em_hbm, midx_v.at[j], me_out, base + j * _CH)
               for j in range(_NCHUNK)]
        )
        bufs = (rows0_v, rows1_v)
        sems = (sem0, sem1)
        tbl0, idx0, _, _ = chunks[0]
        gat0 = pltpu.async_copy(tbl0.at[idx0], bufs[0], sems[0])
        pending = gat0
        for k, (_, _, out_ref, off) in enumerate(chunks):
            pending.wait()
            if k + 1 < len(chunks):
                tbl, idx, _, _ = chunks[k + 1]
                pending = pltpu.async_copy(
                    tbl.at[idx], bufs[(k + 1) % 2], sems[(k + 1) % 2])
            pltpu.sync_copy(bufs[k % 2], out_ref.at[pl.ds(off, _CH)])

    return _sc_gather


_BM = 1024


def _mlp_body(ue_ref, me_ref, w1a_ref, w1b_ref, b1_ref, w2_ref, b2_ref,
              wf_ref, bf_ref, out_ref):
    x = jnp.dot(ue_ref[...], w1a_ref[...], preferred_element_type=jnp.float32)
    x = x + jnp.dot(me_ref[...], w1b_ref[...], preferred_element_type=jnp.float32)
    x = jax.nn.relu(x + b1_ref[...])
    x = jnp.dot(x, w2_ref[...], preferred_element_type=jnp.float32)
    x = jax.nn.relu(x + b2_ref[...])
    x = jnp.dot(x, wf_ref[...], preferred_element_type=jnp.float32)
    out_ref[...] = jax.nn.sigmoid(x + bf_ref[...])


def _mlp(ue, me, W1, b1, W2, b2, Wf, bf):
    h1, h2 = W1.shape[1], W2.shape[1]
    grid = (BATCH // _BM,)
    return pl.pallas_call(
        _mlp_body,
        grid=grid,
        in_specs=[
            pl.BlockSpec((_BM, D), lambda i: (i, 0)),
            pl.BlockSpec((_BM, D), lambda i: (i, 0)),
            pl.BlockSpec((D, h1), lambda i: (0, 0)),
            pl.BlockSpec((D, h1), lambda i: (0, 0)),
            pl.BlockSpec((1, h1), lambda i: (0, 0)),
            pl.BlockSpec((h1, h2), lambda i: (0, 0)),
            pl.BlockSpec((1, h2), lambda i: (0, 0)),
            pl.BlockSpec((h2, 1), lambda i: (0, 0)),
            pl.BlockSpec((1, 1), lambda i: (0, 0)),
        ],
        out_specs=pl.BlockSpec((_BM, 1), lambda i: (i, 0)),
        out_shape=jax.ShapeDtypeStruct((BATCH, 1), jnp.float32),
    )(ue, me, W1[:D], W1[D:], b1.reshape(1, h1), W2, b2.reshape(1, h2),
      Wf, bf.reshape(1, 1))


def kernel(users, movies, Eu, Em, W1, b1, W2, b2, Wf, bf):
    u2 = users.astype(jnp.int32).reshape(BATCH // _CH, _CH)
    m2 = movies.astype(jnp.int32).reshape(BATCH // _CH, _CH)
    ue, me = _make_sc_gather()(u2, m2, Eu, Em)
    return _mlp(ue, me, W1, b1, W2, b2, Wf, bf)
